# Initial kernel scaffold; baseline (speedup 1.0000x reference)
import jax
import jax.numpy as jnp
from jax.experimental import pallas as pl


def kernel(x, edge_index, W1, b1, W2, b2, W3, b3, g1, be1, g2, be2):
    raise NotImplementedError("write your pallas kernel here")


# speedup vs baseline: 12.0051x; 12.0051x over previous
import functools

import jax
import jax.numpy as jnp
from jax import lax
from jax.experimental import pallas as pl
from jax.experimental.pallas import tpu as pltpu
from jax.experimental.pallas import tpu_sc as plsc

N = 10000
E = 320000
D_IN = 128
D_H = 128
D_OUT = 64

NC = 2
NS = 16
NW = NC * NS
EPW = E // NW
CHUNK = 80
NCHUNK = EPW // CHUNK

N_PAD = 10240
RPT = N_PAD // NS
ZROWS = 128

_F32 = jnp.float32


def _mesh():
    return plsc.VectorSubcoreMesh(core_axis_name="c", subcore_axis_name="s")


def _sc_degree(dst):
    @functools.partial(
        pl.kernel,
        out_type=jax.ShapeDtypeStruct((NC, N_PAD), _F32),
        mesh=_mesh(),
        scratch_types=[
            pltpu.VMEM((CHUNK,), jnp.int32),
            pltpu.VMEM((CHUNK,), _F32),
            pltpu.VMEM((RPT,), _F32),
            pltpu.VMEM_SHARED((N_PAD,), _F32),
        ],
    )
    def k(dst_hbm, out_hbm, didx, ones, zbuf, acc):
        c = lax.axis_index("c")
        s = lax.axis_index("s")
        wid = c * NS + s

        @pl.loop(0, RPT // 16)
        def _(i):
            zbuf[pl.ds(i * 16, 16)] = jnp.zeros((16,), _F32)

        @pl.loop(0, CHUNK // 16)
        def _(i):
            ones[pl.ds(i * 16, 16)] = jnp.ones((16,), _F32)

        pltpu.sync_copy(zbuf, acc.at[pl.ds(s * RPT, RPT)])
        plsc.subcore_barrier()

        @pl.loop(0, NCHUNK)
        def _(i):
            base = pl.multiple_of(wid * EPW + i * CHUNK, 8)
            pltpu.sync_copy(dst_hbm.at[pl.ds(base, CHUNK)], didx)
            pltpu.sync_copy(ones, acc.at[didx], add=True)

        plsc.subcore_barrier()
        pltpu.sync_copy(acc.at[pl.ds(s * RPT, RPT)],
                        out_hbm.at[c, pl.ds(s * RPT, RPT)])

    return k(dst)


def _sc_aggregate(hp, src, dst, d):
    @functools.partial(
        pl.kernel,
        out_type=jax.ShapeDtypeStruct((NC, N_PAD, d), _F32),
        mesh=_mesh(),
        compiler_params=pltpu.CompilerParams(use_tc_tiling_on_sc=False),
        scratch_types=[
            pltpu.VMEM((CHUNK,), jnp.int32),
            pltpu.VMEM((CHUNK,), jnp.int32),
            pltpu.VMEM((CHUNK, d), _F32),
            pltpu.VMEM((ZROWS, d), _F32),
            pltpu.VMEM_SHARED((N_PAD, d), _F32),
            pltpu.SemaphoreType.DMA,
        ],
    )
    def k(hp_hbm, src_hbm, dst_hbm, out_hbm, sidx, didx, rows, zbuf, acc, sem):
        c = lax.axis_index("c")
        s = lax.axis_index("s")
        wid = c * NS + s

        @pl.loop(0, ZROWS)
        def _(r):
            row = zbuf.at[r]
            for j in range(d // 16):
                row[pl.ds(j * 16, 16)] = jnp.zeros((16,), _F32)

        @pl.loop(0, RPT // ZROWS)
        def _(j):
            pltpu.sync_copy(zbuf, acc.at[pl.ds(s * RPT + j * ZROWS, ZROWS)])

        plsc.subcore_barrier()

        @pl.loop(0, NCHUNK)
        def _(i):
            base = pl.multiple_of(wid * EPW + i * CHUNK, 8)
            pltpu.sync_copy(src_hbm.at[pl.ds(base, CHUNK)], sidx)
            pltpu.sync_copy(dst_hbm.at[pl.ds(base, CHUNK)], didx)
            pltpu.async_copy(hp_hbm.at[sidx], rows, sem).wait()
            pltpu.sync_copy(rows, acc.at[didx], add=True)

        plsc.subcore_barrier()
        pltpu.sync_copy(acc.at[pl.ds(s * RPT, RPT)],
                        out_hbm.at[c, pl.ds(s * RPT, RPT)])

    return k(hp, src, dst)


def _dinv_col(degp):
    return lax.rsqrt(degp[0, :N] + degp[1, :N] + 1.0)[:, None]


def _tc_first(x, w, degp):
    def body(x_ref, w_ref, degp_ref, out_ref):
        dinv = _dinv_col(degp_ref[...])
        h = jnp.dot(x_ref[...], w_ref[...], preferred_element_type=_F32)
        out_ref[...] = h * dinv

    return pl.pallas_call(
        body, out_shape=jax.ShapeDtypeStruct((N, D_H), _F32)
    )(x, w, degp)


def _tc_mid(degp, aggp, hp, b, g, be, w):
    def body(degp_ref, aggp_ref, hp_ref, b_ref, g_ref, be_ref, w_ref, out_ref):
        dinv = _dinv_col(degp_ref[...])
        t = (aggp_ref[0, :N, :] + aggp_ref[1, :N, :] + hp_ref[...]) * dinv
        t = t + b_ref[...]
        mu = jnp.mean(t, axis=0, keepdims=True)
        var = jnp.mean((t - mu) ** 2, axis=0, keepdims=True)
        t = (t - mu) * lax.rsqrt(var + 1e-5) * g_ref[...] + be_ref[...]
        t = jnp.maximum(t, 0.0)
        out_ref[...] = jnp.dot(t, w_ref[...], preferred_element_type=_F32) * dinv

    d_next = w.shape[1]
    return pl.pallas_call(
        body, out_shape=jax.ShapeDtypeStruct((N, d_next), _F32)
    )(degp, aggp, hp, b, g, be, w)


def _tc_final(degp, aggp, hp, b):
    def body(degp_ref, aggp_ref, hp_ref, b_ref, out_ref):
        dinv = _dinv_col(degp_ref[...])
        t = (aggp_ref[0, :N, :] + aggp_ref[1, :N, :] + hp_ref[...]) * dinv
        t = t + b_ref[...]
        m = jnp.max(t, axis=1, keepdims=True)
        e = jnp.exp(t - m)
        out_ref[...] = e / jnp.sum(e, axis=1, keepdims=True)

    return pl.pallas_call(
        body, out_shape=jax.ShapeDtypeStruct((N, D_OUT), _F32)
    )(degp, aggp, hp, b)


def kernel(x, edge_index, W1, b1, W2, b2, W3, b3, g1, be1, g2, be2):
    src = edge_index[0]
    dst = edge_index[1]

    degp = _sc_degree(dst)
    h1p = _tc_first(x, W1, degp)
    agg1 = _sc_aggregate(h1p, src, dst, D_H)
    h2p = _tc_mid(degp, agg1, h1p, b1, g1, be1, W2)
    agg2 = _sc_aggregate(h2p, src, dst, D_H)
    h3p = _tc_mid(degp, agg2, h2p, b2, g2, be2, W3)
    agg3 = _sc_aggregate(h3p, src, dst, D_OUT)
    return _tc_final(degp, agg3, h3p, b3)


# speedup vs baseline: 33.9345x; 2.8267x over previous
import functools

import jax
import jax.numpy as jnp
from jax import lax
from jax.experimental import pallas as pl
from jax.experimental.pallas import tpu as pltpu
from jax.experimental.pallas import tpu_sc as plsc

N = 10000
E = 320000
D_IN = 128
D_H = 128
D_OUT = 64

NC = 2
NS = 16
NW = NC * NS
EPW = E // NW

CH_D = 80
NCH_D = EPW // CH_D
N_PAD = 10240
RPT = N_PAD // NS

CH_A = 40
NCH_A = EPW // CH_A
NBUF = 5
RING = NCH_A // NBUF
N_ACC = 10112
RPA = N_ACC // NS

_F32 = jnp.float32


def _mesh():
    return plsc.VectorSubcoreMesh(core_axis_name="c", subcore_axis_name="s")


def _sc_degree(dstd):
    @functools.partial(
        pl.kernel,
        out_type=jax.ShapeDtypeStruct((NC, N_PAD), _F32),
        mesh=_mesh(),
        scratch_types=[
            pltpu.VMEM((NCH_D, CH_D), jnp.int32),
            pltpu.VMEM((CH_D,), _F32),
            pltpu.VMEM((RPT,), _F32),
            pltpu.VMEM_SHARED((N_PAD,), _F32),
        ],
    )
    def k(dst_hbm, out_hbm, didx2, ones, zbuf, acc):
        c = lax.axis_index("c")
        s = lax.axis_index("s")
        wid = c * NS + s

        pltpu.sync_copy(dst_hbm.at[wid], didx2)

        @pl.loop(0, RPT // 16)
        def _(i):
            zbuf[pl.ds(i * 16, 16)] = jnp.zeros((16,), _F32)

        @pl.loop(0, CH_D // 16)
        def _(i):
            ones[pl.ds(i * 16, 16)] = jnp.ones((16,), _F32)

        pltpu.sync_copy(zbuf, acc.at[pl.ds(s * RPT, RPT)])
        plsc.subcore_barrier()

        @pl.loop(0, NCH_D)
        def _(i):
            pltpu.sync_copy(ones, acc.at[didx2.at[i]], add=True)

        plsc.subcore_barrier()
        pltpu.sync_copy(acc.at[pl.ds(s * RPT, RPT)],
                        out_hbm.at[c, pl.ds(s * RPT, RPT)])

    return k(dstd)


def _sc_aggregate(hp, zeros, srca, dsta, d):
    @functools.partial(
        pl.kernel,
        out_type=jax.ShapeDtypeStruct((NC, N_ACC, d), _F32),
        mesh=_mesh(),
        compiler_params=pltpu.CompilerParams(use_tc_tiling_on_sc=False),
        scratch_types=[
            pltpu.VMEM((NCH_A, CH_A), jnp.int32),
            pltpu.VMEM((NCH_A, CH_A), jnp.int32),
        ]
        + [pltpu.VMEM((CH_A, d), _F32) for _ in range(NBUF)]
        + [pltpu.VMEM_SHARED((N_ACC, d), _F32)]
        + [pltpu.SemaphoreType.DMA for _ in range(NBUF)],
    )
    def k(hp_hbm, z_hbm, src_hbm, dst_hbm, out_hbm, sidx2, didx2, *rest):
        rows = rest[:NBUF]
        acc = rest[NBUF]
        gsem = rest[NBUF + 1:]
        c = lax.axis_index("c")
        s = lax.axis_index("s")
        wid = c * NS + s

        pltpu.sync_copy(src_hbm.at[wid], sidx2)
        pltpu.sync_copy(dst_hbm.at[wid], didx2)
        pltpu.sync_copy(z_hbm.at[pl.ds(s * RPA, RPA)],
                        acc.at[pl.ds(s * RPA, RPA)])
        plsc.subcore_barrier()

        for b in range(NBUF):
            pltpu.async_copy(hp_hbm.at[sidx2.at[b]], rows[b], gsem[b])

        @pl.loop(0, RING - 1)
        def _(j):
            i0 = j * NBUF
            for b in range(NBUF):
                pltpu.make_async_copy(
                    hp_hbm.at[sidx2.at[i0 + b]], rows[b], gsem[b]).wait()
                pltpu.sync_copy(rows[b], acc.at[didx2.at[i0 + b]], add=True)
                pltpu.async_copy(
                    hp_hbm.at[sidx2.at[i0 + NBUF + b]], rows[b], gsem[b])

        for b in range(NBUF):
            i = (RING - 1) * NBUF + b
            pltpu.make_async_copy(
                hp_hbm.at[sidx2.at[i]], rows[b], gsem[b]).wait()
            pltpu.sync_copy(rows[b], acc.at[didx2.at[i]], add=True)

        plsc.subcore_barrier()
        pltpu.sync_copy(acc.at[pl.ds(s * RPA, RPA)],
                        out_hbm.at[c, pl.ds(s * RPA, RPA)])

    return k(hp, zeros, srca, dsta)


def _dinv_col(degp):
    return lax.rsqrt(degp[0, :N] + degp[1, :N] + 1.0)[:, None]


def _tc_first(x, w, degp):
    def body(x_ref, w_ref, degp_ref, out_ref):
        dinv = _dinv_col(degp_ref[...])
        h = jnp.dot(x_ref[...], w_ref[...], preferred_element_type=_F32)
        out_ref[...] = h * dinv

    return pl.pallas_call(
        body, out_shape=jax.ShapeDtypeStruct((N, D_H), _F32)
    )(x, w, degp)


def _tc_mid(degp, aggp, hp, b, g, be, w):
    def body(degp_ref, aggp_ref, hp_ref, b_ref, g_ref, be_ref, w_ref, out_ref):
        dinv = _dinv_col(degp_ref[...])
        t = (aggp_ref[0, :N, :] + aggp_ref[1, :N, :] + hp_ref[...]) * dinv
        t = t + b_ref[...]
        mu = jnp.mean(t, axis=0, keepdims=True)
        var = jnp.mean((t - mu) ** 2, axis=0, keepdims=True)
        t = (t - mu) * lax.rsqrt(var + 1e-5) * g_ref[...] + be_ref[...]
        t = jnp.maximum(t, 0.0)
        out_ref[...] = jnp.dot(t, w_ref[...], preferred_element_type=_F32) * dinv

    d_next = w.shape[1]
    return pl.pallas_call(
        body, out_shape=jax.ShapeDtypeStruct((N, d_next), _F32)
    )(degp, aggp, hp, b, g, be, w)


def _tc_final(degp, aggp, hp, b):
    def body(degp_ref, aggp_ref, hp_ref, b_ref, out_ref):
        dinv = _dinv_col(degp_ref[...])
        t = (aggp_ref[0, :N, :] + aggp_ref[1, :N, :] + hp_ref[...]) * dinv
        t = t + b_ref[...]
        m = jnp.max(t, axis=1, keepdims=True)
        e = jnp.exp(t - m)
        out_ref[...] = e / jnp.sum(e, axis=1, keepdims=True)

    return pl.pallas_call(
        body, out_shape=jax.ShapeDtypeStruct((N, D_OUT), _F32)
    )(degp, aggp, hp, b)


def kernel(x, edge_index, W1, b1, W2, b2, W3, b3, g1, be1, g2, be2):
    src = edge_index[0]
    dst = edge_index[1]
    dstd = dst.reshape(NW, NCH_D, CH_D)
    srca = src.reshape(NW, NCH_A, CH_A)
    dsta = dst.reshape(NW, NCH_A, CH_A)
    zeros_h = jnp.zeros((N_ACC, D_H), _F32)
    zeros_o = zeros_h[:, :D_OUT]

    degp = _sc_degree(dstd)
    h1p = _tc_first(x, W1, degp)
    agg1 = _sc_aggregate(h1p, zeros_h, srca, dsta, D_H)
    h2p = _tc_mid(degp, agg1, h1p, b1, g1, be1, W2)
    agg2 = _sc_aggregate(h2p, zeros_h, srca, dsta, D_H)
    h3p = _tc_mid(degp, agg2, h2p, b2, g2, be2, W3)
    agg3 = _sc_aggregate(h3p, zeros_o, srca, dsta, D_OUT)
    return _tc_final(degp, agg3, h3p, b3)
